# Initial kernel scaffold; baseline (speedup 1.0000x reference)
import jax
import jax.numpy as jnp
from jax.experimental import pallas as pl


def kernel(edge_index, h, e, norm):
    raise NotImplementedError("write your pallas kernel here")


# speedup vs baseline: 6.6708x; 6.6708x over previous
import functools

import jax
import jax.numpy as jnp
from jax import lax
from jax.experimental import pallas as pl
from jax.experimental.pallas import tpu as pltpu
from jax.experimental.pallas import tpu_sc as plsc

N = 10000
D = 128
E = 320000

NC = 2
NS = 16
NW = NC * NS
K = 128
NCHUNK = E // K
BASE_CH = NCHUNK // NW
EXTRA = NCHUNK - BASE_CH * NW
NP = 10240
RPT = NP // NS


def _scale_body(h_ref, n_ref, o_ref):
    o_ref[...] = h_ref[...] * n_ref[...]


def _combine_body(p_ref, n_ref, o_ref):
    s = (p_ref[0] + p_ref[1]) * n_ref[...]
    o_ref[:, :D] = s
    o_ref[:, D:] = s


def _sc_body(edge_hbm, hn_hbm, zero_hbm, out_hbm, src_v, dst_v, rows_v, acc_sh, sem):
    cid = lax.axis_index("c")
    sid = lax.axis_index("s")
    wid = sid * NC + cid

    pltpu.sync_copy(zero_hbm.at[pl.ds(sid * RPT, RPT)],
                    acc_sh.at[pl.ds(sid * RPT, RPT)])
    plsc.subcore_barrier()

    nch = BASE_CH + jnp.where(wid < EXTRA, 1, 0)

    def body(j, carry):
        base = (j * NW + wid) * K
        pltpu.sync_copy(edge_hbm.at[0, pl.ds(base, K)], src_v)
        pltpu.sync_copy(edge_hbm.at[1, pl.ds(base, K)], dst_v)
        pltpu.async_copy(hn_hbm.at[src_v], rows_v, sem).wait()
        pltpu.sync_copy(rows_v, acc_sh.at[dst_v], add=True)
        return carry

    lax.fori_loop(0, nch, body, 0)
    plsc.subcore_barrier()
    pltpu.sync_copy(acc_sh.at[pl.ds(sid * RPT, RPT)],
                    out_hbm.at[cid, pl.ds(sid * RPT, RPT)])


def kernel(edge_index, h, e, norm):
    hn = pl.pallas_call(
        _scale_body,
        grid=(5,),
        in_specs=[pl.BlockSpec((N // 5, D), lambda i: (i, 0)),
                  pl.BlockSpec((N // 5, 1), lambda i: (i, 0))],
        out_specs=pl.BlockSpec((N // 5, D), lambda i: (i, 0)),
        out_shape=jax.ShapeDtypeStruct((N, D), jnp.float32),
    )(h, norm)

    zeros = jnp.zeros((NP, D), jnp.float32)

    mesh = plsc.VectorSubcoreMesh(core_axis_name="c", subcore_axis_name="s")
    sc_fn = pl.kernel(
        _sc_body,
        out_type=jax.ShapeDtypeStruct((NC, NP, D), jnp.float32),
        mesh=mesh,
        scratch_types=[
            pltpu.VMEM((K,), jnp.int32),
            pltpu.VMEM((K,), jnp.int32),
            pltpu.VMEM((K, D), jnp.float32),
            pltpu.VMEM_SHARED((NP, D), jnp.float32),
            pltpu.SemaphoreType.DMA,
        ],
    )
    partials = sc_fn(edge_index, hn, zeros)

    out = pl.pallas_call(
        _combine_body,
        grid=(5,),
        in_specs=[pl.BlockSpec((NC, N // 5, D), lambda i: (0, i, 0)),
                  pl.BlockSpec((N // 5, 1), lambda i: (i, 0))],
        out_specs=pl.BlockSpec((N // 5, 2 * D), lambda i: (i, 0)),
        out_shape=jax.ShapeDtypeStruct((N, 2 * D), jnp.float32),
    )(partials, norm)
    return out, e


# speedup vs baseline: 9.4600x; 1.4181x over previous
import functools

import jax
import jax.numpy as jnp
from jax import lax
from jax.experimental import pallas as pl
from jax.experimental.pallas import tpu as pltpu
from jax.experimental.pallas import tpu_sc as plsc

N = 10000
D = 128
E = 320000

NC = 2
NS = 16
NW = NC * NS
K = 64
NCHUNK = E // K
BASE_CH = NCHUNK // NW
EXTRA = NCHUNK - BASE_CH * NW
NP = 10112
RPT = NP // NS
ZR = 10240

DEPTH = 6
NROWS = 5
NITER = BASE_CH // DEPTH


def _scale_body(h_ref, n_ref, o_ref, z_ref):
    o_ref[...] = h_ref[...] * n_ref[...]
    z_ref[...] = jnp.zeros_like(z_ref)


def _combine_body(p_ref, n_ref, o_ref):
    s = (p_ref[0] + p_ref[1]) * n_ref[...]
    o_ref[:, :D] = s
    o_ref[:, D:] = s


def _sc_body(src_hbm, dst_hbm, hn_hbm, zero_hbm, out_hbm, *scratch):
    idx = scratch[0:DEPTH]
    rows = scratch[DEPTH:DEPTH + NROWS]
    acc_sh = scratch[DEPTH + NROWS]
    sems = scratch[DEPTH + NROWS + 1:]
    isem = sems[0:DEPTH]
    gsem = sems[DEPTH:DEPTH + NROWS]
    ssem = sems[DEPTH + NROWS:DEPTH + 2 * NROWS]

    cid = lax.axis_index("c")
    sid = lax.axis_index("s")
    wid = sid * NC + cid

    pltpu.sync_copy(zero_hbm.at[pl.ds(sid * RPT, RPT)],
                    acc_sh.at[pl.ds(sid * RPT, RPT)])
    plsc.subcore_barrier()

    def chunk_base(k, i):
        return ((k * DEPTH + i) * NW + wid) * K

    def load_idx(k, i):
        a = pltpu.async_copy(src_hbm.at[pl.ds(chunk_base(k, i), K)],
                             idx[i].at[0], isem[i])
        b = pltpu.async_copy(dst_hbm.at[pl.ds(chunk_base(k, i), K)],
                             idx[i].at[1], isem[i])
        return a, b

    def body(k, carry):
        di = [load_idx(k, i) for i in range(DEPTH)]
        dg = [None] * DEPTH
        ds_ = [None] * DEPTH
        for i in range(DEPTH):
            di[i][0].wait()
            di[i][1].wait()
            if i >= NROWS:
                dg[i - NROWS].wait()
                ds_[i - NROWS] = pltpu.async_copy(
                    rows[i % NROWS], acc_sh.at[idx[i - NROWS].at[1]],
                    ssem[i % NROWS], add=True)
                ds_[i - NROWS].wait()
            dg[i] = pltpu.async_copy(hn_hbm.at[idx[i].at[0]],
                                     rows[i % NROWS], gsem[i % NROWS])
        for i in range(DEPTH - NROWS, DEPTH):
            dg[i].wait()
            ds_[i] = pltpu.async_copy(rows[i % NROWS], acc_sh.at[idx[i].at[1]],
                                      ssem[i % NROWS], add=True)
        for i in range(DEPTH - NROWS, DEPTH):
            ds_[i].wait()
        return carry

    lax.fori_loop(0, NITER, body, 0)

    @pl.when(wid < EXTRA)
    def _tail():
        base = (BASE_CH * NW + wid) * K
        pltpu.sync_copy(src_hbm.at[pl.ds(base, K)], idx[0].at[0])
        pltpu.sync_copy(dst_hbm.at[pl.ds(base, K)], idx[0].at[1])
        pltpu.async_copy(hn_hbm.at[idx[0].at[0]], rows[0], gsem[0]).wait()
        pltpu.sync_copy(rows[0], acc_sh.at[idx[0].at[1]], add=True)

    plsc.subcore_barrier()
    pltpu.sync_copy(acc_sh.at[pl.ds(sid * RPT, RPT)],
                    out_hbm.at[cid, pl.ds(sid * RPT, RPT)])


def kernel(edge_index, h, e, norm):
    src = edge_index[0]
    dst = edge_index[1]

    hn, zeros = pl.pallas_call(
        _scale_body,
        grid=(5,),
        in_specs=[pl.BlockSpec((N // 5, D), lambda i: (i, 0)),
                  pl.BlockSpec((N // 5, 1), lambda i: (i, 0))],
        out_specs=[pl.BlockSpec((N // 5, D), lambda i: (i, 0)),
                   pl.BlockSpec((ZR // 5, D), lambda i: (i, 0))],
        out_shape=[jax.ShapeDtypeStruct((N, D), jnp.float32),
                   jax.ShapeDtypeStruct((ZR, D), jnp.float32)],
    )(h, norm)

    mesh = plsc.VectorSubcoreMesh(core_axis_name="c", subcore_axis_name="s")
    sc_fn = pl.kernel(
        _sc_body,
        out_type=jax.ShapeDtypeStruct((NC, NP, D), jnp.float32),
        mesh=mesh,
        scratch_types=(
            [pltpu.VMEM((2, K), jnp.int32) for _ in range(DEPTH)]
            + [pltpu.VMEM((K, D), jnp.float32) for _ in range(NROWS)]
            + [pltpu.VMEM_SHARED((NP, D), jnp.float32)]
            + [pltpu.SemaphoreType.DMA for _ in range(DEPTH + 2 * NROWS)]
        ),
    )
    partials = sc_fn(src, dst, hn, zeros)

    out = pl.pallas_call(
        _combine_body,
        grid=(5,),
        in_specs=[pl.BlockSpec((NC, N // 5, D), lambda i: (0, i, 0)),
                  pl.BlockSpec((N // 5, 1), lambda i: (i, 0))],
        out_specs=pl.BlockSpec((N // 5, 2 * D), lambda i: (i, 0)),
        out_shape=jax.ShapeDtypeStruct((N, 2 * D), jnp.float32),
    )(partials, norm)
    return out, e


# speedup vs baseline: 11.2611x; 1.1904x over previous
import functools

import jax
import jax.numpy as jnp
from jax import lax
from jax.experimental import pallas as pl
from jax.experimental.pallas import tpu as pltpu
from jax.experimental.pallas import tpu_sc as plsc

N = 10000
D = 128
E = 320000

NC = 2
NS = 16
NW = NC * NS
K = 128
NCHUNK = E // K
BASE_CH = NCHUNK // NW
EXTRA = NCHUNK - BASE_CH * NW
NP = 10112
RPT = NP // NS
ZR = 10240

DEPTH = 6
NROWS = 6
NITER = BASE_CH // DEPTH


def _scale_body(h_ref, n_ref, o_ref, z_ref):
    o_ref[...] = (h_ref[...] * n_ref[...]).astype(jnp.bfloat16)
    z_ref[...] = jnp.zeros_like(z_ref)


def _combine_body(p_ref, n_ref, o_ref):
    s = (p_ref[0].astype(jnp.float32) + p_ref[1].astype(jnp.float32)) * n_ref[...]
    o_ref[:, :D] = s
    o_ref[:, D:] = s


def _sc_body(src_hbm, dst_hbm, hn_hbm, zero_hbm, out_hbm, *scratch):
    idx = scratch[0:DEPTH]
    rows = scratch[DEPTH:DEPTH + NROWS]
    acc_sh = scratch[DEPTH + NROWS]
    sems = scratch[DEPTH + NROWS + 1:]
    isem = sems[0:DEPTH]
    gsem = sems[DEPTH:DEPTH + NROWS]
    ssem = sems[DEPTH + NROWS:DEPTH + 2 * NROWS]

    cid = lax.axis_index("c")
    sid = lax.axis_index("s")
    wid = sid * NC + cid

    pltpu.sync_copy(zero_hbm.at[pl.ds(sid * RPT, RPT)],
                    acc_sh.at[pl.ds(sid * RPT, RPT)])
    plsc.subcore_barrier()

    def chunk_base(k, i):
        return ((k * DEPTH + i) * NW + wid) * K

    def load_idx(k, i):
        a = pltpu.async_copy(src_hbm.at[pl.ds(chunk_base(k, i), K)],
                             idx[i].at[0], isem[i])
        b = pltpu.async_copy(dst_hbm.at[pl.ds(chunk_base(k, i), K)],
                             idx[i].at[1], isem[i])
        return a, b

    def body(k, carry):
        di = [load_idx(k, i) for i in range(DEPTH)]
        dg = [None] * DEPTH
        ds_ = [None] * DEPTH
        for i in range(DEPTH):
            di[i][0].wait()
            di[i][1].wait()
            dg[i] = pltpu.async_copy(hn_hbm.at[idx[i].at[0]], rows[i],
                                     gsem[i])
        for i in range(DEPTH):
            dg[i].wait()
            ds_[i] = pltpu.async_copy(rows[i], acc_sh.at[idx[i].at[1]],
                                      ssem[i], add=True)
        for i in range(DEPTH):
            ds_[i].wait()
        return carry

    lax.fori_loop(0, NITER, body, 0)

    @pl.when(wid < EXTRA)
    def _tail():
        base = (BASE_CH * NW + wid) * K
        pltpu.sync_copy(src_hbm.at[pl.ds(base, K)], idx[0].at[0])
        pltpu.sync_copy(dst_hbm.at[pl.ds(base, K)], idx[0].at[1])
        pltpu.async_copy(hn_hbm.at[idx[0].at[0]], rows[0], gsem[0]).wait()
        pltpu.sync_copy(rows[0], acc_sh.at[idx[0].at[1]], add=True)

    plsc.subcore_barrier()
    pltpu.sync_copy(acc_sh.at[pl.ds(sid * RPT, RPT)],
                    out_hbm.at[cid, pl.ds(sid * RPT, RPT)])


def kernel(edge_index, h, e, norm):
    src = edge_index[0]
    dst = edge_index[1]

    hn, zeros = pl.pallas_call(
        _scale_body,
        grid=(5,),
        in_specs=[pl.BlockSpec((N // 5, D), lambda i: (i, 0)),
                  pl.BlockSpec((N // 5, 1), lambda i: (i, 0))],
        out_specs=[pl.BlockSpec((N // 5, D), lambda i: (i, 0)),
                   pl.BlockSpec((ZR // 5, D), lambda i: (i, 0))],
        out_shape=[jax.ShapeDtypeStruct((N, D), jnp.bfloat16),
                   jax.ShapeDtypeStruct((ZR, D), jnp.bfloat16)],
    )(h, norm)

    mesh = plsc.VectorSubcoreMesh(core_axis_name="c", subcore_axis_name="s")
    sc_fn = pl.kernel(
        _sc_body,
        out_type=jax.ShapeDtypeStruct((NC, NP, D), jnp.bfloat16),
        mesh=mesh,
        compiler_params=pltpu.CompilerParams(use_tc_tiling_on_sc=False),
        scratch_types=(
            [pltpu.VMEM((2, K), jnp.int32) for _ in range(DEPTH)]
            + [pltpu.VMEM((K, D), jnp.bfloat16) for _ in range(NROWS)]
            + [pltpu.VMEM_SHARED((NP, D), jnp.bfloat16)]
            + [pltpu.SemaphoreType.DMA for _ in range(DEPTH + 2 * NROWS)]
        ),
    )
    partials = sc_fn(src, dst, hn, zeros)

    out = pl.pallas_call(
        _combine_body,
        grid=(5,),
        in_specs=[pl.BlockSpec((NC, N // 5, D), lambda i: (0, i, 0)),
                  pl.BlockSpec((N // 5, 1), lambda i: (i, 0))],
        out_specs=pl.BlockSpec((N // 5, 2 * D), lambda i: (i, 0)),
        out_shape=jax.ShapeDtypeStruct((N, 2 * D), jnp.float32),
    )(partials, norm)
    return out, e


# speedup vs baseline: 11.8367x; 1.0511x over previous
import functools

import jax
import jax.numpy as jnp
from jax import lax
from jax.experimental import pallas as pl
from jax.experimental.pallas import tpu as pltpu
from jax.experimental.pallas import tpu_sc as plsc

N = 10000
D = 128
E = 320000

NC = 2
NS = 16
NW = NC * NS
K = 128
NCHUNK = E // K
BASE_CH = NCHUNK // NW
EXTRA = NCHUNK - BASE_CH * NW
NP = 10112
RPT = NP // NS
ZR = 10240

DEPTH = 10
NROWS = 10
NITER = BASE_CH // DEPTH
EPI = BASE_CH - NITER * DEPTH


def _scale_body(h_ref, n_ref, o_ref, z_ref):
    o_ref[...] = (h_ref[...] * n_ref[...]).astype(jnp.bfloat16)
    z_ref[...] = jnp.zeros_like(z_ref)


def _combine_body(p_ref, n_ref, o_ref):
    s = (p_ref[0].astype(jnp.float32) + p_ref[1].astype(jnp.float32)) * n_ref[...]
    o_ref[:, :D] = s
    o_ref[:, D:] = s


def _sc_body(src_hbm, dst_hbm, hn_hbm, zero_hbm, out_hbm, *scratch):
    idx = scratch[0:DEPTH]
    rows = scratch[DEPTH:DEPTH + NROWS]
    acc_sh = scratch[DEPTH + NROWS]
    sems = scratch[DEPTH + NROWS + 1:]
    isem = sems[0:DEPTH]
    gsem = sems[DEPTH:DEPTH + NROWS]
    ssem = sems[DEPTH + NROWS:DEPTH + 2 * NROWS]

    cid = lax.axis_index("c")
    sid = lax.axis_index("s")
    wid = sid * NC + cid

    pltpu.sync_copy(zero_hbm.at[pl.ds(sid * RPT, RPT)],
                    acc_sh.at[pl.ds(sid * RPT, RPT)])
    plsc.subcore_barrier()

    def load_idx(chunk, i):
        base = (chunk * NW + wid) * K
        a = pltpu.async_copy(src_hbm.at[pl.ds(base, K)],
                             idx[i].at[0], isem[i])
        b = pltpu.async_copy(dst_hbm.at[pl.ds(base, K)],
                             idx[i].at[1], isem[i])
        return a, b

    def run_group(chunk0, n):
        di = [load_idx(chunk0 + i, i) for i in range(n)]
        dg = [None] * n
        ds_ = [None] * n
        for i in range(n):
            di[i][0].wait()
            di[i][1].wait()
            dg[i] = pltpu.async_copy(hn_hbm.at[idx[i].at[0]], rows[i],
                                     gsem[i])
        for i in range(n):
            dg[i].wait()
            ds_[i] = pltpu.async_copy(rows[i], acc_sh.at[idx[i].at[1]],
                                      ssem[i], add=True)
        for i in range(n):
            ds_[i].wait()

    def body(k, carry):
        run_group(k * DEPTH, DEPTH)
        return carry

    lax.fori_loop(0, NITER, body, 0)
    run_group(NITER * DEPTH, EPI)

    @pl.when(wid < EXTRA)
    def _tail():
        base = (BASE_CH * NW + wid) * K
        pltpu.sync_copy(src_hbm.at[pl.ds(base, K)], idx[0].at[0])
        pltpu.sync_copy(dst_hbm.at[pl.ds(base, K)], idx[0].at[1])
        pltpu.async_copy(hn_hbm.at[idx[0].at[0]], rows[0], gsem[0]).wait()
        pltpu.sync_copy(rows[0], acc_sh.at[idx[0].at[1]], add=True)

    plsc.subcore_barrier()
    pltpu.sync_copy(acc_sh.at[pl.ds(sid * RPT, RPT)],
                    out_hbm.at[cid, pl.ds(sid * RPT, RPT)])


def kernel(edge_index, h, e, norm):
    src = edge_index[0]
    dst = edge_index[1]

    hn, zeros = pl.pallas_call(
        _scale_body,
        grid=(5,),
        in_specs=[pl.BlockSpec((N // 5, D), lambda i: (i, 0)),
                  pl.BlockSpec((N // 5, 1), lambda i: (i, 0))],
        out_specs=[pl.BlockSpec((N // 5, D), lambda i: (i, 0)),
                   pl.BlockSpec((ZR // 5, D), lambda i: (i, 0))],
        out_shape=[jax.ShapeDtypeStruct((N, D), jnp.bfloat16),
                   jax.ShapeDtypeStruct((ZR, D), jnp.bfloat16)],
    )(h, norm)

    mesh = plsc.VectorSubcoreMesh(core_axis_name="c", subcore_axis_name="s")
    sc_fn = pl.kernel(
        _sc_body,
        out_type=jax.ShapeDtypeStruct((NC, NP, D), jnp.bfloat16),
        mesh=mesh,
        compiler_params=pltpu.CompilerParams(use_tc_tiling_on_sc=False),
        scratch_types=(
            [pltpu.VMEM((2, K), jnp.int32) for _ in range(DEPTH)]
            + [pltpu.VMEM((K, D), jnp.bfloat16) for _ in range(NROWS)]
            + [pltpu.VMEM_SHARED((NP, D), jnp.bfloat16)]
            + [pltpu.SemaphoreType.DMA for _ in range(DEPTH + 2 * NROWS)]
        ),
    )
    partials = sc_fn(src, dst, hn, zeros)

    out = pl.pallas_call(
        _combine_body,
        grid=(5,),
        in_specs=[pl.BlockSpec((NC, N // 5, D), lambda i: (0, i, 0)),
                  pl.BlockSpec((N // 5, 1), lambda i: (i, 0))],
        out_specs=pl.BlockSpec((N // 5, 2 * D), lambda i: (i, 0)),
        out_shape=jax.ShapeDtypeStruct((N, 2 * D), jnp.float32),
    )(partials, norm)
    return out, e


# speedup vs baseline: 12.5703x; 1.0620x over previous
import functools

import jax
import jax.numpy as jnp
from jax import lax
from jax.experimental import pallas as pl
from jax.experimental.pallas import tpu as pltpu
from jax.experimental.pallas import tpu_sc as plsc

N = 10000
D = 128
E = 320000

NC = 2
NS = 16
NW = NC * NS
K = 128
NCHUNK = E // K
BASE_CH = NCHUNK // NW
EXTRA = NCHUNK - BASE_CH * NW
NP = 10112
RPT = NP // NS
ZR = 10240

DEPTH = 10
NROWS = 10
NITER = BASE_CH // DEPTH
EPI = BASE_CH - NITER * DEPTH


def _scale_body(h_ref, n_ref, o_ref, z_ref):
    o_ref[...] = (h_ref[...] * n_ref[...]).astype(jnp.bfloat16)
    z_ref[...] = jnp.zeros_like(z_ref)


def _combine_body(p_ref, n_ref, o_ref):
    s = (p_ref[0].astype(jnp.float32) + p_ref[1].astype(jnp.float32)) * n_ref[...]
    o_ref[:, :D] = s
    o_ref[:, D:] = s


def _sc_body(edge_hbm, hn_hbm, zero_hbm, out_hbm, *scratch):
    idx = scratch[0:DEPTH]
    rows = scratch[DEPTH:DEPTH + NROWS]
    acc_sh = scratch[DEPTH + NROWS]
    sems = scratch[DEPTH + NROWS + 1:]
    isem = sems[0:DEPTH]
    gsem = sems[DEPTH:DEPTH + NROWS]
    ssem = sems[DEPTH + NROWS:DEPTH + 2 * NROWS]

    cid = lax.axis_index("c")
    sid = lax.axis_index("s")
    wid = sid * NC + cid

    pltpu.sync_copy(zero_hbm.at[pl.ds(sid * RPT, RPT)],
                    acc_sh.at[pl.ds(sid * RPT, RPT)])
    plsc.subcore_barrier()

    def load_idx(chunk, i):
        base = (chunk * NW + wid) * K
        return pltpu.async_copy(edge_hbm.at[:, pl.ds(base, K)],
                                idx[i], isem[i])

    def run_group(chunk0, n):
        di = [load_idx(chunk0 + i, i) for i in range(n)]
        dg = [None] * n
        ds_ = [None] * n
        for i in range(n):
            di[i].wait()
            dg[i] = pltpu.async_copy(hn_hbm.at[idx[i].at[0]], rows[i],
                                     gsem[i])
        for i in range(n):
            dg[i].wait()
            ds_[i] = pltpu.async_copy(rows[i], acc_sh.at[idx[i].at[1]],
                                      ssem[i], add=True)
        for i in range(n):
            ds_[i].wait()

    def body(k, carry):
        run_group(k * DEPTH, DEPTH)
        return carry

    lax.fori_loop(0, NITER, body, 0)
    run_group(NITER * DEPTH, EPI)

    @pl.when(wid < EXTRA)
    def _tail():
        base = (BASE_CH * NW + wid) * K
        pltpu.sync_copy(edge_hbm.at[:, pl.ds(base, K)], idx[0])
        pltpu.async_copy(hn_hbm.at[idx[0].at[0]], rows[0], gsem[0]).wait()
        pltpu.sync_copy(rows[0], acc_sh.at[idx[0].at[1]], add=True)

    plsc.subcore_barrier()
    pltpu.sync_copy(acc_sh.at[pl.ds(sid * RPT, RPT)],
                    out_hbm.at[cid, pl.ds(sid * RPT, RPT)])


def kernel(edge_index, h, e, norm):
    hn, zeros = pl.pallas_call(
        _scale_body,
        grid=(5,),
        in_specs=[pl.BlockSpec((N // 5, D), lambda i: (i, 0)),
                  pl.BlockSpec((N // 5, 1), lambda i: (i, 0))],
        out_specs=[pl.BlockSpec((N // 5, D), lambda i: (i, 0)),
                   pl.BlockSpec((ZR // 5, D), lambda i: (i, 0))],
        out_shape=[jax.ShapeDtypeStruct((N, D), jnp.bfloat16),
                   jax.ShapeDtypeStruct((ZR, D), jnp.bfloat16)],
    )(h, norm)

    mesh = plsc.VectorSubcoreMesh(core_axis_name="c", subcore_axis_name="s")
    sc_fn = pl.kernel(
        _sc_body,
        out_type=jax.ShapeDtypeStruct((NC, NP, D), jnp.bfloat16),
        mesh=mesh,
        compiler_params=pltpu.CompilerParams(use_tc_tiling_on_sc=False),
        scratch_types=(
            [pltpu.VMEM((2, K), jnp.int32) for _ in range(DEPTH)]
            + [pltpu.VMEM((K, D), jnp.bfloat16) for _ in range(NROWS)]
            + [pltpu.VMEM_SHARED((NP, D), jnp.bfloat16)]
            + [pltpu.SemaphoreType.DMA for _ in range(DEPTH + 2 * NROWS)]
        ),
    )
    partials = sc_fn(edge_index, hn, zeros)

    out = pl.pallas_call(
        _combine_body,
        grid=(5,),
        in_specs=[pl.BlockSpec((NC, N // 5, D), lambda i: (0, i, 0)),
                  pl.BlockSpec((N // 5, 1), lambda i: (i, 0))],
        out_specs=pl.BlockSpec((N // 5, 2 * D), lambda i: (i, 0)),
        out_shape=jax.ShapeDtypeStruct((N, 2 * D), jnp.float32),
    )(partials, norm)
    return out, e
